# baseline (speedup 1.0000x reference)
import functools

import jax
import jax.numpy as jnp
from jax import lax
from jax.experimental import pallas as pl
from jax.experimental.pallas import tpu as pltpu
from jax.experimental.pallas import tpu_sc as plsc

N, D, Q = 16384, 64, 512
NBLK = 8192
INV = 1.0 / 16384.0

SC_NC, SC_NS = 2, 16
SC_NW = SC_NC * SC_NS
QPW = Q // SC_NW
L = 16


def _match_argmin_body(xt_ref, xst_ref, out_ref, acc_ref):
    blk = pl.program_id(0)
    xqt = xt_ref[...]
    xbt = xst_ref[...]
    qb2t = (xqt + xqt).astype(jnp.bfloat16)
    dbt = xbt.astype(jnp.bfloat16)
    g2 = lax.dot_general(qb2t, dbt, (((0,), (0,)), ((), ())),
                         preferred_element_type=jnp.float32)
    sqq = xqt * xqt
    ones_col = jnp.ones((D, 1), jnp.float32)
    qn = lax.dot_general(sqq, ones_col, (((0,), (0,)), ((), ())),
                         preferred_element_type=jnp.float32)
    sq = dbt * dbt
    ones_row = jnp.ones((1, D), jnp.bfloat16)
    xn = lax.dot_general(ones_row, sq, (((1,), (0,)), ((), ())),
                         preferred_element_type=jnp.float32)
    iotaf = lax.broadcasted_iota(jnp.int32, (1, NBLK), 1).astype(
        jnp.float32) * INV + blk.astype(jnp.float32) * (NBLK * INV)
    key = ((qn + xn) - g2) + iotaf
    m = jnp.min(key, axis=1, keepdims=True)

    @pl.when(blk == 0)
    def _():
        acc_ref[...] = m

    @pl.when(blk > 0)
    def _():
        acc_ref[...] = jnp.minimum(acc_ref[...], m)

    @pl.when(blk == N // NBLK - 1)
    def _():
        out_ref[...] = acc_ref[...][:, 0]


def _tc_match_argmin(Xt, xst, interpret=False):
    grid = (N // NBLK,)
    return pl.pallas_call(
        _match_argmin_body,
        grid=grid,
        in_specs=[
            pl.BlockSpec((D, Q), lambda i: (0, 0)),
            pl.BlockSpec((D, NBLK), lambda i: (0, i)),
        ],
        out_specs=pl.BlockSpec((Q,), lambda i: (0,)),
        out_shape=jax.ShapeDtypeStruct((Q,), jnp.float32),
        scratch_shapes=[pltpu.VMEM((Q, 1), jnp.float32)],
        interpret=interpret,
    )(Xt, xst)


def _sc_gather(yst_flat, mkey):
    mesh = plsc.VectorSubcoreMesh(core_axis_name="c", subcore_axis_name="s")

    @functools.partial(
        pl.kernel,
        mesh=mesh,
        compiler_params=pltpu.CompilerParams(use_tc_tiling_on_sc=False),
        out_type=jax.ShapeDtypeStruct((3, Q), jnp.float32),
        scratch_types=[
            pltpu.VMEM((QPW,), jnp.float32),
            pltpu.VMEM((3, QPW), jnp.int32),
            pltpu.VMEM((3, QPW), jnp.float32),
            pltpu.SemaphoreType.DMA,
        ],
    )
    def k(ys_hbm, mkey_hbm, out_hbm, mk_v, idx3_v, rows_v, sem):
        wid = lax.axis_index("s") * SC_NC + lax.axis_index("c")
        base = wid * QPW
        pltpu.sync_copy(mkey_hbm.at[pl.ds(base, QPW)], mk_v)
        v = jnp.clip((mk_v[...] * 16384.0).astype(jnp.int32), 0, N - 1)
        for c in range(3):
            idx3_v[c, :] = v + c * N
            pltpu.async_copy(ys_hbm.at[idx3_v.at[c]], rows_v.at[c], sem)
        pltpu.make_async_copy(ys_hbm.at[idx3_v.at[0]], rows_v.at[0], sem).wait()
        pltpu.make_async_copy(ys_hbm.at[idx3_v.at[1]], rows_v.at[1], sem).wait()
        pltpu.make_async_copy(ys_hbm.at[idx3_v.at[2]], rows_v.at[2], sem).wait()
        pltpu.sync_copy(rows_v, out_hbm.at[:, pl.ds(base, QPW)])

    return k(yst_flat, mkey)


def kernel(X, xs, ys):
    m = _tc_match_argmin(X.T, xs.T)
    out3q = _sc_gather(ys.T.reshape(-1), m)
    return out3q.T


# speedup vs baseline: 1.1737x; 1.1737x over previous
import functools

import jax
import jax.numpy as jnp
from jax import lax
from jax.experimental import pallas as pl
from jax.experimental.pallas import tpu as pltpu
from jax.experimental.pallas import tpu_sc as plsc

N, D, Q = 16384, 64, 512
NBLK = 4096
INV = 1.0 / 16384.0

SC_NC, SC_NS = 2, 16
SC_NW = SC_NC * SC_NS
QPW = Q // SC_NW
L = 16


def _match_argmin_body(xt_ref, xst_ref, out_ref, acc_ref):
    blk = pl.program_id(0)
    xqt = xt_ref[...]
    xbt = xst_ref[...]

    sqq = xqt * xqt
    qn = jnp.sum(sqq, axis=0, keepdims=True)
    qn_hi = jnp.floor(qn * (1.0 / 256.0))
    qn_lo = qn - 256.0 * qn_hi
    c = lambda v: jnp.full((1, Q), v, jnp.float32)
    lhs = jnp.concatenate(
        [xqt * (-2048.0), qn_hi, qn_lo, c(262144.0), c(1024.0),
         c(128.0), c(1.0)], axis=0).astype(jnp.bfloat16)

    sq = xbt * xbt
    xn = jnp.sum(sq, axis=0, keepdims=True)
    xn_hi = jnp.floor(xn * (1.0 / 256.0))
    xn_lo = xn - 256.0 * xn_hi
    ii = lax.broadcasted_iota(jnp.int32, (1, NBLK), 1)
    nsub = jnp.bitwise_and(ii, 1023)
    n_hi = jnp.right_shift(nsub, 7).astype(jnp.float32)
    n_lo = jnp.bitwise_and(nsub, 127).astype(jnp.float32)
    d = lambda v: jnp.full((1, NBLK), v, jnp.float32)
    rhs = jnp.concatenate(
        [xbt, d(262144.0), d(1024.0), xn_hi, xn_lo, n_hi, n_lo],
        axis=0).astype(jnp.bfloat16)

    key = lax.dot_general(lhs, rhs, (((0,), (0,)), ((), ())),
                          preferred_element_type=jnp.float32)

    cands = []
    for s in range(NBLK // 1024):
        ms = jnp.min(key[:, s * 1024:(s + 1) * 1024], axis=1,
                     keepdims=True)
        base = blk.astype(jnp.float32) * (NBLK * INV) + s * (1024 * INV)
        cands.append(jnp.where(ms < 1024.0, ms * INV + base, 2.0))
    m = cands[0]
    for t in cands[1:]:
        m = jnp.minimum(m, t)

    @pl.when(blk == 0)
    def _():
        acc_ref[...] = m

    @pl.when(blk > 0)
    def _():
        acc_ref[...] = jnp.minimum(acc_ref[...], m)

    @pl.when(blk == N // NBLK - 1)
    def _():
        out_ref[...] = acc_ref[...][:, 0]


def _tc_match_argmin(Xt, xst, interpret=False):
    grid = (N // NBLK,)
    return pl.pallas_call(
        _match_argmin_body,
        grid=grid,
        in_specs=[
            pl.BlockSpec((D, Q), lambda i: (0, 0)),
            pl.BlockSpec((D, NBLK), lambda i: (0, i)),
        ],
        out_specs=pl.BlockSpec((Q,), lambda i: (0,)),
        out_shape=jax.ShapeDtypeStruct((Q,), jnp.float32),
        scratch_shapes=[pltpu.VMEM((Q, 1), jnp.float32)],
        interpret=interpret,
    )(Xt, xst)


def _sc_gather(yst_flat, mkey):
    mesh = plsc.VectorSubcoreMesh(core_axis_name="c", subcore_axis_name="s")

    @functools.partial(
        pl.kernel,
        mesh=mesh,
        compiler_params=pltpu.CompilerParams(use_tc_tiling_on_sc=False),
        out_type=jax.ShapeDtypeStruct((3, Q), jnp.float32),
        scratch_types=[
            pltpu.VMEM((QPW,), jnp.float32),
            pltpu.VMEM((3, QPW), jnp.int32),
            pltpu.VMEM((3, QPW), jnp.float32),
            pltpu.SemaphoreType.DMA,
        ],
    )
    def k(ys_hbm, mkey_hbm, out_hbm, mk_v, idx3_v, rows_v, sem):
        wid = lax.axis_index("s") * SC_NC + lax.axis_index("c")
        base = wid * QPW
        pltpu.sync_copy(mkey_hbm.at[pl.ds(base, QPW)], mk_v)
        v = jnp.clip((mk_v[...] * 16384.0).astype(jnp.int32), 0, N - 1)
        for c in range(3):
            idx3_v[c, :] = v + c * N
            pltpu.async_copy(ys_hbm.at[idx3_v.at[c]], rows_v.at[c], sem)
        pltpu.make_async_copy(ys_hbm.at[idx3_v.at[0]], rows_v.at[0], sem).wait()
        pltpu.make_async_copy(ys_hbm.at[idx3_v.at[1]], rows_v.at[1], sem).wait()
        pltpu.make_async_copy(ys_hbm.at[idx3_v.at[2]], rows_v.at[2], sem).wait()
        pltpu.sync_copy(rows_v, out_hbm.at[:, pl.ds(base, QPW)])

    return k(yst_flat, mkey)


def kernel(X, xs, ys):
    m = _tc_match_argmin(X.T, xs.T)
    out3q = _sc_gather(ys.T.reshape(-1), m)
    return out3q.T


# speedup vs baseline: 1.1819x; 1.0070x over previous
import functools

import jax
import jax.numpy as jnp
from jax import lax
from jax.experimental import pallas as pl
from jax.experimental.pallas import tpu as pltpu
from jax.experimental.pallas import tpu_sc as plsc

N, D, Q = 16384, 64, 512
NBLK = 8192
INV = 1.0 / 16384.0

SC_NC, SC_NS = 2, 16
SC_NW = SC_NC * SC_NS
QPW = Q // SC_NW
L = 16


def _match_argmin_body(xt_ref, xst_ref, out_ref, acc_ref):
    blk = pl.program_id(0)
    xqt = xt_ref[...]
    xbt = xst_ref[...]

    sqq = xqt * xqt
    qn = jnp.sum(sqq, axis=0, keepdims=True)
    qn_hi = jnp.floor(qn * (1.0 / 256.0))
    qn_lo = qn - 256.0 * qn_hi
    c = lambda v: jnp.full((1, Q), v, jnp.float32)
    lhs = jnp.concatenate(
        [xqt * (-2048.0), qn_hi, qn_lo, c(262144.0), c(1024.0),
         c(128.0), c(1.0)], axis=0).astype(jnp.bfloat16)

    sq = xbt * xbt
    xn = jnp.sum(sq, axis=0, keepdims=True)
    xn_hi = jnp.floor(xn * (1.0 / 256.0))
    xn_lo = xn - 256.0 * xn_hi
    ii = lax.broadcasted_iota(jnp.int32, (1, NBLK), 1)
    nsub = jnp.bitwise_and(ii, 1023)
    n_hi = jnp.right_shift(nsub, 7).astype(jnp.float32)
    n_lo = jnp.bitwise_and(nsub, 127).astype(jnp.float32)
    d = lambda v: jnp.full((1, NBLK), v, jnp.float32)
    rhs = jnp.concatenate(
        [xbt, d(262144.0), d(1024.0), xn_hi, xn_lo, n_hi, n_lo],
        axis=0).astype(jnp.bfloat16)

    key = lax.dot_general(lhs, rhs, (((0,), (0,)), ((), ())),
                          preferred_element_type=jnp.float32)

    cands = []
    for s in range(NBLK // 1024):
        ms = jnp.min(key[:, s * 1024:(s + 1) * 1024], axis=1,
                     keepdims=True)
        base = blk.astype(jnp.float32) * (NBLK * INV) + s * (1024 * INV)
        cands.append(jnp.where(ms < 1024.0, ms * INV + base, 2.0))
    m = cands[0]
    for t in cands[1:]:
        m = jnp.minimum(m, t)

    @pl.when(blk == 0)
    def _():
        acc_ref[...] = m

    @pl.when(blk > 0)
    def _():
        acc_ref[...] = jnp.minimum(acc_ref[...], m)

    @pl.when(blk == N // NBLK - 1)
    def _():
        out_ref[...] = acc_ref[...][:, 0]


def _tc_match_argmin(Xt, xst, interpret=False):
    grid = (N // NBLK,)
    return pl.pallas_call(
        _match_argmin_body,
        grid=grid,
        in_specs=[
            pl.BlockSpec((D, Q), lambda i: (0, 0)),
            pl.BlockSpec((D, NBLK), lambda i: (0, i)),
        ],
        out_specs=pl.BlockSpec((Q,), lambda i: (0,)),
        out_shape=jax.ShapeDtypeStruct((Q,), jnp.float32),
        scratch_shapes=[pltpu.VMEM((Q, 1), jnp.float32)],
        interpret=interpret,
    )(Xt, xst)


def _sc_gather(yst_flat, mkey):
    mesh = plsc.VectorSubcoreMesh(core_axis_name="c", subcore_axis_name="s")

    @functools.partial(
        pl.kernel,
        mesh=mesh,
        compiler_params=pltpu.CompilerParams(use_tc_tiling_on_sc=False),
        out_type=jax.ShapeDtypeStruct((3, Q), jnp.float32),
        scratch_types=[
            pltpu.VMEM((QPW,), jnp.float32),
            pltpu.VMEM((3, QPW), jnp.int32),
            pltpu.VMEM((3, QPW), jnp.float32),
            pltpu.SemaphoreType.DMA,
        ],
    )
    def k(ys_hbm, mkey_hbm, out_hbm, mk_v, idx3_v, rows_v, sem):
        wid = lax.axis_index("s") * SC_NC + lax.axis_index("c")
        base = wid * QPW
        pltpu.sync_copy(mkey_hbm.at[pl.ds(base, QPW)], mk_v)
        v = jnp.clip((mk_v[...] * 16384.0).astype(jnp.int32), 0, N - 1)
        for c in range(3):
            idx3_v[c, :] = v + c * N
            pltpu.async_copy(ys_hbm.at[idx3_v.at[c]], rows_v.at[c], sem)
        pltpu.make_async_copy(ys_hbm.at[idx3_v.at[0]], rows_v.at[0], sem).wait()
        pltpu.make_async_copy(ys_hbm.at[idx3_v.at[1]], rows_v.at[1], sem).wait()
        pltpu.make_async_copy(ys_hbm.at[idx3_v.at[2]], rows_v.at[2], sem).wait()
        pltpu.sync_copy(rows_v, out_hbm.at[:, pl.ds(base, QPW)])

    return k(yst_flat, mkey)


def kernel(X, xs, ys):
    m = _tc_match_argmin(X.T, xs.T)
    out3q = _sc_gather(ys.T.reshape(-1), m)
    return out3q.T
